# baseline (speedup 1.0000x reference)
import functools

import jax
import jax.numpy as jnp
from jax import lax
from jax.experimental import pallas as pl
from jax.experimental.pallas import tpu as pltpu
from jax.experimental.pallas import tpu_sc as plsc

K_PER = 256
Q_MIN = 0.5
S_B = 1.0
HUBER_SCALE = 2.0
E_DEN_OFF = 1.0
PAYLOAD_REL_THR = 0.1

_NC = 2
_NS = 16
_L = 16


def _atanh(x):
    return 0.5 * jnp.log((1.0 + x) / (1.0 - x))


_STAGE = 6


def _sc_argmax_body(beta_hbm, tid_hbm, cc0_hbm, cc1_hbm,
                    ba_out, x0_out, x1_out,
                    tkey, targ, bv_ref, tv_ref, mk_ref, mi_ref,
                    shk, shi, idxb, kbuf, x0b, x1b, sem,
                    *, N, HPW):
    c = lax.axis_index("c")
    s = lax.axis_index("s")
    base = c * (N // 2) + s * HPW
    lane = jnp.arange(_L, dtype=jnp.int32)

    for i in range(K_PER * _L // _L):
        tkey[pl.ds(i * _L, _L)] = jnp.full((_L,), -1.0, jnp.float32)
        targ[pl.ds(i * _L, _L)] = jnp.zeros((_L,), jnp.int32)

    pltpu.sync_copy(beta_hbm.at[pl.ds(base, HPW)], bv_ref)
    pltpu.sync_copy(tid_hbm.at[pl.ds(base, HPW)], tv_ref)

    if _STAGE >= 2:
        for i in range(HPW // _L):
            bv = bv_ref[pl.ds(i * _L, _L)]
            bv = jnp.minimum(jnp.maximum(bv, 1e-6), 1.0 - 1e-4)
            tv = tv_ref[pl.ds(i * _L, _L)]
            valid = tv >= 0
            slot = jnp.where(valid, tv, 0) * _L + lane
            cur = plsc.load_gather(tkey, [slot])
            m = valid & (bv > cur)
            plsc.store_scatter(tkey, [slot], bv, mask=m)
            hidx = jnp.full((_L,), base + i * _L, jnp.int32) + lane
            plsc.store_scatter(targ, [slot], hidx, mask=m)

    if _STAGE >= 3:
        for j in range(K_PER // _L):
            jbase = j * _L * _L
            idx0 = jbase + lane * _L
            ka = plsc.load_gather(tkey, [idx0])
            ia = plsc.load_gather(targ, [idx0])
            for l in range(1, _L):
                kv = plsc.load_gather(tkey, [idx0 + l])
                iv = plsc.load_gather(targ, [idx0 + l])
                better = (kv > ka) | ((kv == ka) & (iv < ia))
                ka = jnp.where(better, kv, ka)
                ia = jnp.where(better, iv, ia)
            mk_ref[pl.ds(j * _L, _L)] = ka
            mi_ref[pl.ds(j * _L, _L)] = ia

    if _STAGE >= 4:
        pltpu.sync_copy(mk_ref, shk.at[s])
        pltpu.sync_copy(mi_ref, shi.at[s])
        plsc.subcore_barrier()

        pltpu.sync_copy(shk.at[0, pl.ds(s * _L, _L)], kbuf)
        pltpu.sync_copy(shi.at[0, pl.ds(s * _L, _L)], idxb)
        ka = kbuf[...]
        ia = idxb[...]
        for w in range(1, _NS):
            pltpu.sync_copy(shk.at[w, pl.ds(s * _L, _L)], kbuf)
            pltpu.sync_copy(shi.at[w, pl.ds(s * _L, _L)], idxb)
            kv = kbuf[...]
            iv = idxb[...]
            better = (kv > ka) | ((kv == ka) & (iv < ia))
            ka = jnp.where(better, kv, ka)
            ia = jnp.where(better, iv, ia)
        kbuf[...] = ka
        idxb[...] = ia

    if _STAGE >= 5:
        pltpu.async_copy(cc0_hbm.at[idxb], x0b, sem).wait()
        pltpu.async_copy(cc1_hbm.at[idxb], x1b, sem).wait()

    out = pl.ds(c * K_PER + s * _L, _L)
    pltpu.sync_copy(kbuf, ba_out.at[out])
    pltpu.sync_copy(x0b, x0_out.at[out])
    pltpu.sync_copy(x1b, x1_out.at[out])


def _sc_argmax(beta, tid, cc0, cc1, n):
    mesh = plsc.VectorSubcoreMesh(core_axis_name="c", subcore_axis_name="s",
                                  num_cores=_NC)
    hpw = n // 2 // _NS
    f = pl.kernel(
        functools.partial(_sc_argmax_body, N=n, HPW=hpw),
        out_type=[jax.ShapeDtypeStruct((_NC * K_PER,), jnp.float32)] * 3,
        mesh=mesh,
        compiler_params=pltpu.CompilerParams(needs_layout_passes=False),
        scratch_types=[
            pltpu.VMEM((K_PER * _L,), jnp.float32),
            pltpu.VMEM((K_PER * _L,), jnp.int32),
            pltpu.VMEM((hpw,), jnp.float32),
            pltpu.VMEM((hpw,), jnp.int32),
            pltpu.VMEM((K_PER,), jnp.float32),
            pltpu.VMEM((K_PER,), jnp.int32),
            pltpu.VMEM_SHARED((_NS, K_PER), jnp.float32),
            pltpu.VMEM_SHARED((_NS, K_PER), jnp.int32),
            pltpu.VMEM((_L,), jnp.int32),
            pltpu.VMEM((_L,), jnp.float32),
            pltpu.VMEM((_L,), jnp.float32),
            pltpu.VMEM((_L,), jnp.float32),
            pltpu.SemaphoreType.DMA,
        ],
    )
    return f(beta, tid, cc0, cc1)


def _tc_kernel(rs_ref, beta_ref, cc_ref, pe_ref, pp_ref, pt_ref, pid_ref,
               tidx_ref, te_ref, tt_ref,
               ba_row_ref, xa0_row_ref, xa1_row_ref,
               ba_col_ref, xa0_col_ref, xa1_col_ref,
               out_ref,
               t8_ref, qaeff_ref, acc_ref, accp_ref, ssum_ref,
               *, B, K, NB, N):
    b = pl.program_id(0)
    off = jnp.where(b * B >= rs_ref[1], K_PER, 0)
    ksl = pl.ds(off, K_PER)

    @pl.when(b == 0)
    def _prep():
        ba_c = ba_col_ref[...]
        ex_c = (ba_c > 0.0).astype(jnp.float32)
        bac = ba_c * ex_c
        atc = _atanh(bac)
        qa_c = atc * atc + Q_MIN
        thr_c = PAYLOAD_REL_THR * bac
        zero = jnp.zeros_like(ba_c)
        t8_ref[...] = jnp.concatenate(
            [xa0_col_ref[...], xa1_col_ref[...], qa_c, thr_c,
             zero, zero, zero, zero], axis=1)
        ba_r = ba_row_ref[...]
        ex_r = (ba_r > 0.0).astype(jnp.float32)
        atr = _atanh(ba_r * ex_r)
        qaeff_ref[...] = (atr * atr + Q_MIN) * ex_r
        acc_ref[...] = jnp.zeros((1, K), jnp.float32)
        accp_ref[...] = jnp.zeros((2, K), jnp.float32)
        ssum_ref[0] = 0.0
        ssum_ref[1] = 0.0
        ssum_ref[2] = 0.0

    tid = tidx_ref[:, 0:1]
    beta = jnp.clip(beta_ref[:, 0:1], 1e-6, 1.0 - 1e-4)
    x0 = cc_ref[:, 0:1]
    x1 = cc_ref[:, 1:2]
    kio = jax.lax.broadcasted_iota(jnp.int32, (1, K_PER), 1)
    mf = (tid == kio).astype(jnp.float32)

    g8 = jax.lax.dot_general(
        mf, t8_ref[ksl, :], (((1,), (0,)), ((), ())),
        preferred_element_type=jnp.float32,
        precision=jax.lax.Precision.HIGHEST)
    xg0 = g8[:, 0:1]
    xg1 = g8[:, 1:2]
    qag = g8[:, 2:3]
    thrg = g8[:, 3:4]

    at = _atanh(beta)
    q = at * at + Q_MIN
    dg0 = x0 - xg0
    dg1 = x1 - xg1
    d2g = dg0 * dg0 + dg1 * dg1
    rwg = jnp.maximum(1.0 - jnp.sqrt(d2g + 1e-9), 0.0)
    attdiag = q * qag * (d2g - rwg)

    dx0 = x0 - xa0_row_ref[0:1, ksl]
    dx1 = x1 - xa1_row_ref[0:1, ksl]
    d2 = dx0 * dx0 + dx1 * dx1
    rw = jnp.maximum(1.0 - jnp.sqrt(d2 + 1e-9), 0.0)
    acc_ref[0:1, ksl] += jnp.sum(q * (qaeff_ref[0:1, ksl] * rw),
                                 axis=0, keepdims=True)

    te = te_ref[:, 0:1]
    ew = jnp.where(te > 10.0, 1.0, (te / 10.0 + 0.1) / 1.1)
    ste = jnp.sqrt(te + 0.001)
    l = jnp.abs(te - pe_ref[:, 0:1]) / (ste + E_DEN_OFF)
    delta = ste * HUBER_SCALE
    el = jnp.where(l <= delta, 0.5 * l * l, delta * (l - 0.5 * delta))
    dp0 = tt_ref[:, 1:2] - pp_ref[:, 0:1]
    dp1 = tt_ref[:, 2:3] - pp_ref[:, 1:2]
    posl = (dp0 * dp0 + dp1 * dp1) / 100.0
    dtim = tt_ref[:, 0:1] * 1e9 - pt_ref[:, 0:1]
    timl = dtim * dtim
    cls = (1e-8 / 6.0) * jnp.sum(pid_ref[...] * pid_ref[...], axis=1,
                                 keepdims=True)
    wsum = (el + posl + timl + cls) * ew

    pwv = jnp.where(beta > thrg, beta, 0.0)
    pw2 = jnp.concatenate([pwv, pwv * wsum], axis=1)
    p2 = jax.lax.dot_general(
        pw2, mf, (((0,), (0,)), ((), ())),
        preferred_element_type=jnp.float32,
        precision=jax.lax.Precision.HIGHEST)
    accp_ref[:, ksl] += p2

    nmask = tid < 0
    ssum_ref[0] += jnp.sum(jnp.where(nmask, beta, 0.0))
    ssum_ref[1] += jnp.sum(nmask.astype(jnp.float32))
    ssum_ref[2] += jnp.sum(attdiag)

    @pl.when(b == NB - 1)
    def _fin():
        ba = ba_row_ref[...]
        exists = (ba > 0.0).astype(jnp.float32)
        n_obj = jnp.maximum(jnp.sum(exists), 1.0)
        l_minb = jnp.sum(exists * (1.0 - ba)) / n_obj
        l_pay = jnp.sum(exists * accp_ref[1:2, :]
                        / (accp_ref[0:1, :] + 1e-9)) / n_obj
        pair = (jnp.sum(acc_ref[...]) + ssum_ref[2]) / N
        l_noise = S_B * ssum_ref[0] / jnp.maximum(ssum_ref[1], 1.0)
        out_ref[...] = (pair + l_minb + l_noise + l_pay).reshape(1, 1)


def kernel(pred_beta, pred_ccoords, pred_energy, pred_pos, pred_time,
           pred_id, t_idx, t_energy, t_pos, t_time, t_pid, rowsplits):
    n = pred_beta.shape[0]
    n_events = rowsplits.shape[0] - 1
    k_tot = n_events * K_PER
    B = 1024
    NB = n // B
    tt = jnp.concatenate([t_time, t_pos], axis=1)

    ba, xa0, xa1 = _sc_argmax(pred_beta[:, 0], t_idx[:, 0],
                              pred_ccoords[:, 0], pred_ccoords[:, 1], n)

    hspec = lambda c: pl.BlockSpec((B, c), lambda b: (b, 0))
    rowspec = pl.BlockSpec((1, k_tot), lambda b: (0, 0))
    colspec = pl.BlockSpec((k_tot, 1), lambda b: (0, 0))
    out = pl.pallas_call(
        functools.partial(_tc_kernel, B=B, K=k_tot, NB=NB, N=n),
        grid=(NB,),
        in_specs=[
            pl.BlockSpec(memory_space=pltpu.SMEM),
            hspec(1), hspec(2), hspec(1), hspec(2), hspec(1), hspec(6),
            hspec(1), hspec(1), hspec(3),
            rowspec, rowspec, rowspec,
            colspec, colspec, colspec,
        ],
        out_specs=pl.BlockSpec((1, 1), lambda b: (0, 0)),
        out_shape=jax.ShapeDtypeStruct((1, 1), jnp.float32),
        scratch_shapes=[
            pltpu.VMEM((k_tot, 8), jnp.float32),
            pltpu.VMEM((1, k_tot), jnp.float32),
            pltpu.VMEM((1, k_tot), jnp.float32),
            pltpu.VMEM((2, k_tot), jnp.float32),
            pltpu.SMEM((4,), jnp.float32),
        ],
        compiler_params=pltpu.CompilerParams(
            dimension_semantics=("arbitrary",)),
    )(rowsplits, pred_beta, pred_ccoords, pred_energy, pred_pos, pred_time,
      pred_id, t_idx, t_energy, tt,
      ba.reshape(1, k_tot), xa0.reshape(1, k_tot), xa1.reshape(1, k_tot),
      ba.reshape(k_tot, 1), xa0.reshape(k_tot, 1), xa1.reshape(k_tot, 1))
    return out[0, 0]


# speedup vs baseline: 2.6961x; 2.6961x over previous
import functools

import jax
import jax.numpy as jnp
from jax import lax
from jax.experimental import pallas as pl
from jax.experimental.pallas import tpu as pltpu
from jax.experimental.pallas import tpu_sc as plsc

K_PER = 256
Q_MIN = 0.5
S_B = 1.0
HUBER_SCALE = 2.0
E_DEN_OFF = 1.0
PAYLOAD_REL_THR = 0.1

_NC = 2
_NS = 16
_L = 16


def _atanh(x):
    return 0.5 * jnp.log((1.0 + x) / (1.0 - x))


def _sc_argmax_body(beta_hbm, tid_hbm, cc0_hbm, cc1_hbm,
                    ba_out, x0_out, x1_out, bag_out, xg0_out, xg1_out,
                    tkey, targ, bv_ref, tv_ref, mk_ref, mi_ref,
                    shk, shi, sfk, sf0, sf1,
                    fk_loc, f0_loc, f1_loc,
                    gb_ref, g0_ref, g1_ref,
                    idxb, kbuf, x0b, x1b, sem,
                    *, N, HPW):
    c = lax.axis_index("c")
    s = lax.axis_index("s")
    base = c * (N // 2) + s * HPW
    lane = jnp.arange(_L, dtype=jnp.int32)

    for i in range(K_PER):
        tkey[pl.ds(i * _L, _L)] = jnp.full((_L,), -1.0, jnp.float32)
        targ[pl.ds(i * _L, _L)] = jnp.zeros((_L,), jnp.int32)

    pltpu.sync_copy(beta_hbm.at[pl.ds(base, HPW)], bv_ref)
    pltpu.sync_copy(tid_hbm.at[pl.ds(base, HPW)], tv_ref)

    for i in range(HPW // _L):
        bv = bv_ref[pl.ds(i * _L, _L)]
        bv = jnp.minimum(jnp.maximum(bv, 1e-6), 1.0 - 1e-4)
        tv = tv_ref[pl.ds(i * _L, _L)]
        valid = tv >= 0
        slot = jnp.where(valid, tv, 0) * _L + lane
        cur = plsc.load_gather(tkey, [slot])
        m = valid & (bv > cur)
        plsc.store_scatter(tkey, [slot], bv, mask=m)
        hidx = jnp.full((_L,), base + i * _L, jnp.int32) + lane
        plsc.store_scatter(targ, [slot], hidx, mask=m)

    for j in range(K_PER // _L):
        idx0 = j * _L * _L + lane * _L
        ka = plsc.load_gather(tkey, [idx0])
        ia = plsc.load_gather(targ, [idx0])
        for l in range(1, _L):
            kv = plsc.load_gather(tkey, [idx0 + l])
            iv = plsc.load_gather(targ, [idx0 + l])
            better = (kv > ka) | ((kv == ka) & (iv < ia))
            ka = jnp.where(better, kv, ka)
            ia = jnp.where(better, iv, ia)
        mk_ref[pl.ds(j * _L, _L)] = ka
        mi_ref[pl.ds(j * _L, _L)] = ia

    pltpu.sync_copy(mk_ref, shk.at[s])
    pltpu.sync_copy(mi_ref, shi.at[s])
    plsc.subcore_barrier()

    pltpu.sync_copy(shk.at[0, pl.ds(s * _L, _L)], kbuf)
    pltpu.sync_copy(shi.at[0, pl.ds(s * _L, _L)], idxb)
    ka = kbuf[...]
    ia = idxb[...]
    for w in range(1, _NS):
        pltpu.sync_copy(shk.at[w, pl.ds(s * _L, _L)], kbuf)
        pltpu.sync_copy(shi.at[w, pl.ds(s * _L, _L)], idxb)
        kv = kbuf[...]
        iv = idxb[...]
        better = (kv > ka) | ((kv == ka) & (iv < ia))
        ka = jnp.where(better, kv, ka)
        ia = jnp.where(better, iv, ia)
    kbuf[...] = ka
    idxb[...] = ia

    pltpu.async_copy(cc0_hbm.at[idxb], x0b, sem).wait()
    pltpu.async_copy(cc1_hbm.at[idxb], x1b, sem).wait()

    out = pl.ds(c * K_PER + s * _L, _L)
    pltpu.sync_copy(kbuf, ba_out.at[out])
    pltpu.sync_copy(x0b, x0_out.at[out])
    pltpu.sync_copy(x1b, x1_out.at[out])

    osl = pl.ds(s * _L, _L)
    pltpu.sync_copy(kbuf, sfk.at[osl])
    pltpu.sync_copy(x0b, sf0.at[osl])
    pltpu.sync_copy(x1b, sf1.at[osl])
    plsc.subcore_barrier()

    pltpu.sync_copy(sfk, fk_loc)
    pltpu.sync_copy(sf0, f0_loc)
    pltpu.sync_copy(sf1, f1_loc)

    zero = jnp.zeros((_L,), jnp.float32)
    for i in range(HPW // _L):
        tv = tv_ref[pl.ds(i * _L, _L)]
        valid = tv >= 0
        g = jnp.where(valid, tv, 0)
        bag = plsc.load_gather(fk_loc, [g])
        xg0 = plsc.load_gather(f0_loc, [g])
        xg1 = plsc.load_gather(f1_loc, [g])
        hsl = pl.ds(i * _L, _L)
        gb_ref[hsl] = jnp.where(valid, bag, zero)
        g0_ref[hsl] = jnp.where(valid, xg0, zero)
        g1_ref[hsl] = jnp.where(valid, xg1, zero)

    hout = pl.ds(base, HPW)
    pltpu.sync_copy(gb_ref, bag_out.at[hout])
    pltpu.sync_copy(g0_ref, xg0_out.at[hout])
    pltpu.sync_copy(g1_ref, xg1_out.at[hout])


def _sc_argmax(beta, tid, cc0, cc1, n):
    mesh = plsc.VectorSubcoreMesh(core_axis_name="c", subcore_axis_name="s",
                                  num_cores=_NC)
    hpw = n // 2 // _NS
    k_tot = _NC * K_PER
    f = pl.kernel(
        functools.partial(_sc_argmax_body, N=n, HPW=hpw),
        out_type=[jax.ShapeDtypeStruct((k_tot,), jnp.float32)] * 3
        + [jax.ShapeDtypeStruct((n,), jnp.float32)] * 3,
        mesh=mesh,
        compiler_params=pltpu.CompilerParams(needs_layout_passes=False),
        scratch_types=[
            pltpu.VMEM((K_PER * _L,), jnp.float32),
            pltpu.VMEM((K_PER * _L,), jnp.int32),
            pltpu.VMEM((hpw,), jnp.float32),
            pltpu.VMEM((hpw,), jnp.int32),
            pltpu.VMEM((K_PER,), jnp.float32),
            pltpu.VMEM((K_PER,), jnp.int32),
            pltpu.VMEM_SHARED((_NS, K_PER), jnp.float32),
            pltpu.VMEM_SHARED((_NS, K_PER), jnp.int32),
            pltpu.VMEM_SHARED((K_PER,), jnp.float32),
            pltpu.VMEM_SHARED((K_PER,), jnp.float32),
            pltpu.VMEM_SHARED((K_PER,), jnp.float32),
            pltpu.VMEM((K_PER,), jnp.float32),
            pltpu.VMEM((K_PER,), jnp.float32),
            pltpu.VMEM((K_PER,), jnp.float32),
            pltpu.VMEM((hpw,), jnp.float32),
            pltpu.VMEM((hpw,), jnp.float32),
            pltpu.VMEM((hpw,), jnp.float32),
            pltpu.VMEM((_L,), jnp.int32),
            pltpu.VMEM((_L,), jnp.float32),
            pltpu.VMEM((_L,), jnp.float32),
            pltpu.VMEM((_L,), jnp.float32),
            pltpu.SemaphoreType.DMA,
        ],
    )
    return f(beta, tid, cc0, cc1)


def _tc_kernel(rs_ref, cc_ref, tidx_ref,
               bl_ref, tl_ref, c0l_ref, c1l_ref,
               pel_ref, tel_ref, ptl_ref, ttl_ref,
               pp0l_ref, pp1l_ref, tp0l_ref, tp1l_ref, pidt_ref,
               bag_ref, xg0_ref, xg1_ref,
               ba_row_ref, xa0_row_ref, xa1_row_ref,
               out_ref,
               qaeff_ref, acc_ref, accp_ref, ssum_ref,
               *, B, K, NB, N):
    b = pl.program_id(0)
    off = jnp.where(b * B >= rs_ref[1], K_PER, 0)
    ksl = pl.ds(off, K_PER)

    @pl.when(b == 0)
    def _prep():
        ba_r = ba_row_ref[...]
        ex_r = (ba_r > 0.0).astype(jnp.float32)
        atr = _atanh(ba_r * ex_r)
        qaeff_ref[...] = (atr * atr + Q_MIN) * ex_r
        acc_ref[...] = jnp.zeros((1, K), jnp.float32)
        accp_ref[...] = jnp.zeros((8, K), jnp.float32)
        ssum_ref[0] = 0.0
        ssum_ref[1] = 0.0
        ssum_ref[2] = 0.0

    beta = jnp.clip(bl_ref[...], 1e-6, 1.0 - 1e-4)
    tidl = tl_ref[...]
    valid = (tidl >= 0).astype(jnp.float32)
    at = _atanh(beta)
    q = at * at + Q_MIN

    bag = bag_ref[...]
    atg = _atanh(bag)
    qag = (atg * atg + Q_MIN) * valid
    dg0 = c0l_ref[...] - xg0_ref[...]
    dg1 = c1l_ref[...] - xg1_ref[...]
    d2g = dg0 * dg0 + dg1 * dg1
    rwg = jnp.maximum(1.0 - jnp.sqrt(d2g + 1e-9), 0.0)
    attdiag = q * qag * (d2g - rwg)

    te = tel_ref[...]
    ew = jnp.where(te > 10.0, 1.0, (te / 10.0 + 0.1) / 1.1)
    ste = jnp.sqrt(te + 0.001)
    l = jnp.abs(te - pel_ref[...]) / (ste + E_DEN_OFF)
    delta = ste * HUBER_SCALE
    el = jnp.where(l <= delta, 0.5 * l * l, delta * (l - 0.5 * delta))
    dp0 = tp0l_ref[...] - pp0l_ref[...]
    dp1 = tp1l_ref[...] - pp1l_ref[...]
    posl = (dp0 * dp0 + dp1 * dp1) / 100.0
    dtim = ttl_ref[...] * 1e9 - ptl_ref[...]
    timl = dtim * dtim
    cls = (1e-8 / 6.0) * jnp.sum(pidt_ref[...] * pidt_ref[...], axis=0,
                                 keepdims=True)
    wsum = (el + posl + timl + cls) * ew

    pwv = jnp.where(beta > PAYLOAD_REL_THR * bag, beta, 0.0) * valid
    pw2 = jnp.concatenate([pwv, pwv * wsum], axis=0)

    nmask = (tidl < 0).astype(jnp.float32)
    ssum_ref[0] += jnp.sum(nmask * beta)
    ssum_ref[1] += jnp.sum(nmask)
    ssum_ref[2] += jnp.sum(attdiag)

    x0 = cc_ref[:, 0:1]
    x1 = cc_ref[:, 1:2]
    dx0 = x0 - xa0_row_ref[0:1, ksl]
    dx1 = x1 - xa1_row_ref[0:1, ksl]
    d2 = dx0 * dx0 + dx1 * dx1
    rw = jnp.maximum(1.0 - jnp.sqrt(d2 + 1e-9), 0.0)
    p3 = jax.lax.dot_general(q, rw, (((1,), (0,)), ((), ())),
                             preferred_element_type=jnp.float32)
    acc_ref[0:1, ksl] += p3

    tid = tidx_ref[:, 0:1]
    kio = jax.lax.broadcasted_iota(jnp.int32, (1, K_PER), 1)
    mf = (tid == kio).astype(jnp.float32)
    p2 = jax.lax.dot_general(pw2, mf, (((1,), (0,)), ((), ())),
                             preferred_element_type=jnp.float32)
    accp_ref[0:2, ksl] += p2

    @pl.when(b == NB - 1)
    def _fin():
        ba = ba_row_ref[...]
        exists = (ba > 0.0).astype(jnp.float32)
        n_obj = jnp.maximum(jnp.sum(exists), 1.0)
        l_minb = jnp.sum(exists * (1.0 - ba)) / n_obj
        l_pay = jnp.sum(exists * accp_ref[1:2, :]
                        / (accp_ref[0:1, :] + 1e-9)) / n_obj
        pair = (jnp.sum(qaeff_ref[...] * acc_ref[...]) + ssum_ref[2]) / N
        l_noise = S_B * ssum_ref[0] / jnp.maximum(ssum_ref[1], 1.0)
        out_ref[...] = (pair + l_minb + l_noise + l_pay).reshape(1, 1)


def kernel(pred_beta, pred_ccoords, pred_energy, pred_pos, pred_time,
           pred_id, t_idx, t_energy, t_pos, t_time, t_pid, rowsplits):
    n = pred_beta.shape[0]
    n_events = rowsplits.shape[0] - 1
    k_tot = n_events * K_PER
    B = 1024
    NB = n // B

    cc0 = pred_ccoords[:, 0]
    cc1 = pred_ccoords[:, 1]
    ba, xa0, xa1, bag, xg0, xg1 = _sc_argmax(
        pred_beta[:, 0], t_idx[:, 0], cc0, cc1, n)

    row1 = lambda a: a.reshape(1, n)
    hspec = lambda c: pl.BlockSpec((B, c), lambda b: (b, 0))
    lspec = pl.BlockSpec((1, B), lambda b: (0, b))
    rowspec = pl.BlockSpec((1, k_tot), lambda b: (0, 0))
    out = pl.pallas_call(
        functools.partial(_tc_kernel, B=B, K=k_tot, NB=NB, N=n),
        grid=(NB,),
        in_specs=[
            pl.BlockSpec(memory_space=pltpu.SMEM),
            hspec(2), hspec(1),
        ] + [lspec] * 12 + [
            pl.BlockSpec((6, B), lambda b: (0, b)),
        ] + [lspec] * 3 + [rowspec] * 3,
        out_specs=pl.BlockSpec((1, 1), lambda b: (0, 0)),
        out_shape=jax.ShapeDtypeStruct((1, 1), jnp.float32),
        scratch_shapes=[
            pltpu.VMEM((1, k_tot), jnp.float32),
            pltpu.VMEM((1, k_tot), jnp.float32),
            pltpu.VMEM((8, k_tot), jnp.float32),
            pltpu.SMEM((4,), jnp.float32),
        ],
        compiler_params=pltpu.CompilerParams(
            dimension_semantics=("arbitrary",)),
    )(rowsplits, pred_ccoords, t_idx,
      row1(pred_beta[:, 0]), t_idx[:, 0].reshape(1, n),
      row1(cc0), row1(cc1),
      row1(pred_energy[:, 0]), row1(t_energy[:, 0]),
      row1(pred_time[:, 0]), row1(t_time[:, 0]),
      row1(pred_pos[:, 0]), row1(pred_pos[:, 1]),
      row1(t_pos[:, 0]), row1(t_pos[:, 1]),
      pred_id.T,
      row1(bag), row1(xg0), row1(xg1),
      ba.reshape(1, k_tot), xa0.reshape(1, k_tot), xa1.reshape(1, k_tot))
    return out[0, 0]


# speedup vs baseline: 2.9196x; 1.0829x over previous
import functools

import jax
import jax.numpy as jnp
from jax import lax
from jax.experimental import pallas as pl
from jax.experimental.pallas import tpu as pltpu
from jax.experimental.pallas import tpu_sc as plsc

K_PER = 256
Q_MIN = 0.5
S_B = 1.0
HUBER_SCALE = 2.0
E_DEN_OFF = 1.0
PAYLOAD_REL_THR = 0.1

_NC = 2
_NS = 16
_L = 16


def _atanh(x):
    return 0.5 * jnp.log((1.0 + x) / (1.0 - x))


def _sc_argmax_body(beta_hbm, tid_hbm, cc0_hbm, cc1_hbm,
                    ba_out, x0_out, x1_out, bag_out, xg0_out, xg1_out,
                    tkey, targ, bv_ref, tv_ref, mk_ref, mi_ref,
                    shk, shi, sfk, sf0, sf1,
                    fk_loc, f0_loc, f1_loc,
                    gb_ref, g0_ref, g1_ref,
                    idxb, kbuf, x0b, x1b, sem,
                    *, N, HPW):
    c = lax.axis_index("c")
    s = lax.axis_index("s")
    base = c * (N // 2) + s * HPW
    lane = jnp.arange(_L, dtype=jnp.int32)

    for i in range(K_PER):
        tkey[pl.ds(i * _L, _L)] = jnp.full((_L,), -1.0, jnp.float32)
        targ[pl.ds(i * _L, _L)] = jnp.zeros((_L,), jnp.int32)

    pltpu.sync_copy(beta_hbm.at[pl.ds(base, HPW)], bv_ref)
    pltpu.sync_copy(tid_hbm.at[pl.ds(base, HPW)], tv_ref)

    for i in range(HPW // _L):
        bv = bv_ref[pl.ds(i * _L, _L)]
        bv = jnp.minimum(jnp.maximum(bv, 1e-6), 1.0 - 1e-4)
        tv = tv_ref[pl.ds(i * _L, _L)]
        valid = tv >= 0
        slot = jnp.where(valid, tv, 0) * _L + lane
        cur = plsc.load_gather(tkey, [slot])
        m = valid & (bv > cur)
        plsc.store_scatter(tkey, [slot], bv, mask=m)
        hidx = jnp.full((_L,), base + i * _L, jnp.int32) + lane
        plsc.store_scatter(targ, [slot], hidx, mask=m)

    for j in range(K_PER // _L):
        idx0 = j * _L * _L + lane * _L
        ka = plsc.load_gather(tkey, [idx0])
        ia = plsc.load_gather(targ, [idx0])
        for l in range(1, _L):
            kv = plsc.load_gather(tkey, [idx0 + l])
            iv = plsc.load_gather(targ, [idx0 + l])
            better = (kv > ka) | ((kv == ka) & (iv < ia))
            ka = jnp.where(better, kv, ka)
            ia = jnp.where(better, iv, ia)
        mk_ref[pl.ds(j * _L, _L)] = ka
        mi_ref[pl.ds(j * _L, _L)] = ia

    pltpu.sync_copy(mk_ref, shk.at[s])
    pltpu.sync_copy(mi_ref, shi.at[s])
    plsc.subcore_barrier()

    pltpu.sync_copy(shk.at[0, pl.ds(s * _L, _L)], kbuf)
    pltpu.sync_copy(shi.at[0, pl.ds(s * _L, _L)], idxb)
    ka = kbuf[...]
    ia = idxb[...]
    for w in range(1, _NS):
        pltpu.sync_copy(shk.at[w, pl.ds(s * _L, _L)], kbuf)
        pltpu.sync_copy(shi.at[w, pl.ds(s * _L, _L)], idxb)
        kv = kbuf[...]
        iv = idxb[...]
        better = (kv > ka) | ((kv == ka) & (iv < ia))
        ka = jnp.where(better, kv, ka)
        ia = jnp.where(better, iv, ia)
    kbuf[...] = ka
    idxb[...] = ia

    pltpu.async_copy(cc0_hbm.at[idxb], x0b, sem).wait()
    pltpu.async_copy(cc1_hbm.at[idxb], x1b, sem).wait()

    out = pl.ds(c * K_PER + s * _L, _L)
    pltpu.sync_copy(kbuf, ba_out.at[out])
    pltpu.sync_copy(x0b, x0_out.at[out])
    pltpu.sync_copy(x1b, x1_out.at[out])

    osl = pl.ds(s * _L, _L)
    pltpu.sync_copy(kbuf, sfk.at[osl])
    pltpu.sync_copy(x0b, sf0.at[osl])
    pltpu.sync_copy(x1b, sf1.at[osl])
    plsc.subcore_barrier()

    pltpu.sync_copy(sfk, fk_loc)
    pltpu.sync_copy(sf0, f0_loc)
    pltpu.sync_copy(sf1, f1_loc)

    zero = jnp.zeros((_L,), jnp.float32)
    for i in range(HPW // _L):
        tv = tv_ref[pl.ds(i * _L, _L)]
        valid = tv >= 0
        g = jnp.where(valid, tv, 0)
        bag = plsc.load_gather(fk_loc, [g])
        xg0 = plsc.load_gather(f0_loc, [g])
        xg1 = plsc.load_gather(f1_loc, [g])
        hsl = pl.ds(i * _L, _L)
        gb_ref[hsl] = jnp.where(valid, bag, zero)
        g0_ref[hsl] = jnp.where(valid, xg0, zero)
        g1_ref[hsl] = jnp.where(valid, xg1, zero)

    hout = pl.ds(base, HPW)
    pltpu.sync_copy(gb_ref, bag_out.at[hout])
    pltpu.sync_copy(g0_ref, xg0_out.at[hout])
    pltpu.sync_copy(g1_ref, xg1_out.at[hout])


def _sc_argmax(beta, tid, cc0, cc1, n):
    mesh = plsc.VectorSubcoreMesh(core_axis_name="c", subcore_axis_name="s",
                                  num_cores=_NC)
    hpw = n // 2 // _NS
    k_tot = _NC * K_PER
    f = pl.kernel(
        functools.partial(_sc_argmax_body, N=n, HPW=hpw),
        out_type=[jax.ShapeDtypeStruct((k_tot,), jnp.float32)] * 3
        + [jax.ShapeDtypeStruct((n,), jnp.float32)] * 3,
        mesh=mesh,
        compiler_params=pltpu.CompilerParams(needs_layout_passes=False),
        scratch_types=[
            pltpu.VMEM((K_PER * _L,), jnp.float32),
            pltpu.VMEM((K_PER * _L,), jnp.int32),
            pltpu.VMEM((hpw,), jnp.float32),
            pltpu.VMEM((hpw,), jnp.int32),
            pltpu.VMEM((K_PER,), jnp.float32),
            pltpu.VMEM((K_PER,), jnp.int32),
            pltpu.VMEM_SHARED((_NS, K_PER), jnp.float32),
            pltpu.VMEM_SHARED((_NS, K_PER), jnp.int32),
            pltpu.VMEM_SHARED((K_PER,), jnp.float32),
            pltpu.VMEM_SHARED((K_PER,), jnp.float32),
            pltpu.VMEM_SHARED((K_PER,), jnp.float32),
            pltpu.VMEM((K_PER,), jnp.float32),
            pltpu.VMEM((K_PER,), jnp.float32),
            pltpu.VMEM((K_PER,), jnp.float32),
            pltpu.VMEM((hpw,), jnp.float32),
            pltpu.VMEM((hpw,), jnp.float32),
            pltpu.VMEM((hpw,), jnp.float32),
            pltpu.VMEM((_L,), jnp.int32),
            pltpu.VMEM((_L,), jnp.float32),
            pltpu.VMEM((_L,), jnp.float32),
            pltpu.VMEM((_L,), jnp.float32),
            pltpu.SemaphoreType.DMA,
        ],
    )
    return f(beta, tid, cc0, cc1)


def _tc_kernel(rs_ref, cc_ref, tidx_ref,
               bl_ref, tl_ref, c0l_ref, c1l_ref,
               pel_ref, tel_ref, ptl_ref, ttl_ref,
               pp0l_ref, pp1l_ref, tp0l_ref, tp1l_ref, pidt_ref,
               bag_ref, xg0_ref, xg1_ref,
               ba_row_ref, xa0_row_ref, xa1_row_ref,
               out_ref,
               qaeff_ref, acc_ref, accp_ref, ssum_ref,
               *, B, K, NB, N):
    b = pl.program_id(0)
    off = jnp.where(b * B >= rs_ref[1], K_PER, 0)
    ksl = pl.ds(off, K_PER)

    @pl.when(b == 0)
    def _prep():
        ba_r = ba_row_ref[...]
        ex_r = (ba_r > 0.0).astype(jnp.float32)
        atr = _atanh(ba_r * ex_r)
        qaeff_ref[...] = (atr * atr + Q_MIN) * ex_r
        acc_ref[...] = jnp.zeros((1, K), jnp.float32)
        accp_ref[...] = jnp.zeros((8, K), jnp.float32)
        ssum_ref[0] = 0.0
        ssum_ref[1] = 0.0
        ssum_ref[2] = 0.0

    beta = jnp.clip(bl_ref[...], 1e-6, 1.0 - 1e-4)
    tidl = tl_ref[...]
    valid = (tidl >= 0).astype(jnp.float32)
    at = _atanh(beta)
    q = at * at + Q_MIN

    bag = bag_ref[...]
    atg = _atanh(bag)
    qag = (atg * atg + Q_MIN) * valid
    dg0 = c0l_ref[...] - xg0_ref[...]
    dg1 = c1l_ref[...] - xg1_ref[...]
    d2g = dg0 * dg0 + dg1 * dg1
    rwg = jnp.maximum(1.0 - jnp.sqrt(d2g + 1e-9), 0.0)
    attdiag = q * qag * (d2g - rwg)

    te = tel_ref[...]
    ew = jnp.where(te > 10.0, 1.0, (te / 10.0 + 0.1) / 1.1)
    ste = jnp.sqrt(te + 0.001)
    l = jnp.abs(te - pel_ref[...]) / (ste + E_DEN_OFF)
    delta = ste * HUBER_SCALE
    el = jnp.where(l <= delta, 0.5 * l * l, delta * (l - 0.5 * delta))
    dp0 = tp0l_ref[...] - pp0l_ref[...]
    dp1 = tp1l_ref[...] - pp1l_ref[...]
    posl = (dp0 * dp0 + dp1 * dp1) / 100.0
    dtim = ttl_ref[...] * 1e9 - ptl_ref[...]
    timl = dtim * dtim
    cls = (1e-8 / 6.0) * jnp.sum(pidt_ref[...] * pidt_ref[...], axis=0,
                                 keepdims=True)
    wsum = (el + posl + timl + cls) * ew

    pwv = jnp.where(beta > PAYLOAD_REL_THR * bag, beta, 0.0) * valid
    pw2 = jnp.concatenate([pwv, pwv * wsum], axis=0)

    nmask = (tidl < 0).astype(jnp.float32)
    ssum_ref[0] += jnp.sum(nmask * beta)
    ssum_ref[1] += jnp.sum(nmask)
    ssum_ref[2] += jnp.sum(attdiag)

    x0 = cc_ref[:, 0:1]
    x1 = cc_ref[:, 1:2]
    dx0 = x0 - xa0_row_ref[0:1, ksl]
    dx1 = x1 - xa1_row_ref[0:1, ksl]
    d2 = dx0 * dx0 + dx1 * dx1
    rw = jnp.maximum(1.0 - jnp.sqrt(d2 + 1e-9), 0.0)
    p3 = jax.lax.dot_general(q, rw, (((1,), (0,)), ((), ())),
                             preferred_element_type=jnp.float32)
    acc_ref[0:1, ksl] += p3

    tid = tidx_ref[:, 0:1]
    kio = jax.lax.broadcasted_iota(jnp.int32, (1, K_PER), 1)
    mf = (tid == kio).astype(jnp.float32)
    p2 = jax.lax.dot_general(pw2, mf, (((1,), (0,)), ((), ())),
                             preferred_element_type=jnp.float32)
    accp_ref[0:2, ksl] += p2

    @pl.when(b == NB - 1)
    def _fin():
        ba = ba_row_ref[...]
        exists = (ba > 0.0).astype(jnp.float32)
        n_obj = jnp.maximum(jnp.sum(exists), 1.0)
        l_minb = jnp.sum(exists * (1.0 - ba)) / n_obj
        l_pay = jnp.sum(exists * accp_ref[1:2, :]
                        / (accp_ref[0:1, :] + 1e-9)) / n_obj
        pair = (jnp.sum(qaeff_ref[...] * acc_ref[...]) + ssum_ref[2]) / N
        l_noise = S_B * ssum_ref[0] / jnp.maximum(ssum_ref[1], 1.0)
        out_ref[...] = (pair + l_minb + l_noise + l_pay).reshape(1, 1)


def kernel(pred_beta, pred_ccoords, pred_energy, pred_pos, pred_time,
           pred_id, t_idx, t_energy, t_pos, t_time, t_pid, rowsplits):
    n = pred_beta.shape[0]
    n_events = rowsplits.shape[0] - 1
    k_tot = n_events * K_PER
    B = 2048
    NB = n // B

    cc0 = pred_ccoords[:, 0]
    cc1 = pred_ccoords[:, 1]
    ba, xa0, xa1, bag, xg0, xg1 = _sc_argmax(
        pred_beta[:, 0], t_idx[:, 0], cc0, cc1, n)

    row1 = lambda a: a.reshape(1, n)
    hspec = lambda c: pl.BlockSpec((B, c), lambda b: (b, 0))
    lspec = pl.BlockSpec((1, B), lambda b: (0, b))
    rowspec = pl.BlockSpec((1, k_tot), lambda b: (0, 0))
    out = pl.pallas_call(
        functools.partial(_tc_kernel, B=B, K=k_tot, NB=NB, N=n),
        grid=(NB,),
        in_specs=[
            pl.BlockSpec(memory_space=pltpu.SMEM),
            hspec(2), hspec(1),
        ] + [lspec] * 12 + [
            pl.BlockSpec((6, B), lambda b: (0, b)),
        ] + [lspec] * 3 + [rowspec] * 3,
        out_specs=pl.BlockSpec((1, 1), lambda b: (0, 0)),
        out_shape=jax.ShapeDtypeStruct((1, 1), jnp.float32),
        scratch_shapes=[
            pltpu.VMEM((1, k_tot), jnp.float32),
            pltpu.VMEM((1, k_tot), jnp.float32),
            pltpu.VMEM((8, k_tot), jnp.float32),
            pltpu.SMEM((4,), jnp.float32),
        ],
        compiler_params=pltpu.CompilerParams(
            dimension_semantics=("arbitrary",)),
    )(rowsplits, pred_ccoords, t_idx,
      row1(pred_beta[:, 0]), t_idx[:, 0].reshape(1, n),
      row1(cc0), row1(cc1),
      row1(pred_energy[:, 0]), row1(t_energy[:, 0]),
      row1(pred_time[:, 0]), row1(t_time[:, 0]),
      row1(pred_pos[:, 0]), row1(pred_pos[:, 1]),
      row1(t_pos[:, 0]), row1(t_pos[:, 1]),
      pred_id.T,
      row1(bag), row1(xg0), row1(xg1),
      ba.reshape(1, k_tot), xa0.reshape(1, k_tot), xa1.reshape(1, k_tot))
    return out[0, 0]


# speedup vs baseline: 2.9362x; 1.0057x over previous
import functools

import jax
import jax.numpy as jnp
from jax import lax
from jax.experimental import pallas as pl
from jax.experimental.pallas import tpu as pltpu
from jax.experimental.pallas import tpu_sc as plsc

K_PER = 256
Q_MIN = 0.5
S_B = 1.0
HUBER_SCALE = 2.0
E_DEN_OFF = 1.0
PAYLOAD_REL_THR = 0.1

_NC = 2
_NS = 16
_L = 16


def _atanh(x):
    return 0.5 * jnp.log((1.0 + x) / (1.0 - x))


def _sc_argmax_body(beta_hbm, tid_hbm, cc0_hbm, cc1_hbm,
                    ba_out, x0_out, x1_out, bag_out, xg0_out, xg1_out,
                    tkey, targ, bv_ref, tv_ref, mk_ref, mi_ref,
                    shk, shi, sfk, sf0, sf1,
                    fk_loc, f0_loc, f1_loc,
                    gb_ref, g0_ref, g1_ref,
                    idxb, kbuf, x0b, x1b, sem,
                    *, N, HPW):
    c = lax.axis_index("c")
    s = lax.axis_index("s")
    base = c * (N // 2) + s * HPW
    lane = jnp.arange(_L, dtype=jnp.int32)

    for i in range(K_PER):
        tkey[pl.ds(i * _L, _L)] = jnp.full((_L,), -1.0, jnp.float32)
        targ[pl.ds(i * _L, _L)] = jnp.zeros((_L,), jnp.int32)

    pltpu.sync_copy(beta_hbm.at[pl.ds(base, HPW)], bv_ref)
    pltpu.sync_copy(tid_hbm.at[pl.ds(base, HPW)], tv_ref)

    for i in range(HPW // _L):
        bv = bv_ref[pl.ds(i * _L, _L)]
        bv = jnp.minimum(jnp.maximum(bv, 1e-6), 1.0 - 1e-4)
        tv = tv_ref[pl.ds(i * _L, _L)]
        valid = tv >= 0
        slot = jnp.where(valid, tv, 0) * _L + lane
        cur = plsc.load_gather(tkey, [slot])
        m = valid & (bv > cur)
        plsc.store_scatter(tkey, [slot], bv, mask=m)
        hidx = jnp.full((_L,), base + i * _L, jnp.int32) + lane
        plsc.store_scatter(targ, [slot], hidx, mask=m)

    for j in range(K_PER // _L):
        idx0 = j * _L * _L + lane * _L
        ka = plsc.load_gather(tkey, [idx0])
        ia = plsc.load_gather(targ, [idx0])
        for l in range(1, _L):
            kv = plsc.load_gather(tkey, [idx0 + l])
            iv = plsc.load_gather(targ, [idx0 + l])
            better = (kv > ka) | ((kv == ka) & (iv < ia))
            ka = jnp.where(better, kv, ka)
            ia = jnp.where(better, iv, ia)
        mk_ref[pl.ds(j * _L, _L)] = ka
        mi_ref[pl.ds(j * _L, _L)] = ia

    pltpu.sync_copy(mk_ref, shk.at[s])
    pltpu.sync_copy(mi_ref, shi.at[s])
    plsc.subcore_barrier()

    pltpu.sync_copy(shk.at[0, pl.ds(s * _L, _L)], kbuf)
    pltpu.sync_copy(shi.at[0, pl.ds(s * _L, _L)], idxb)
    ka = kbuf[...]
    ia = idxb[...]
    for w in range(1, _NS):
        pltpu.sync_copy(shk.at[w, pl.ds(s * _L, _L)], kbuf)
        pltpu.sync_copy(shi.at[w, pl.ds(s * _L, _L)], idxb)
        kv = kbuf[...]
        iv = idxb[...]
        better = (kv > ka) | ((kv == ka) & (iv < ia))
        ka = jnp.where(better, kv, ka)
        ia = jnp.where(better, iv, ia)
    kbuf[...] = ka
    idxb[...] = ia

    pltpu.async_copy(cc0_hbm.at[idxb], x0b, sem).wait()
    pltpu.async_copy(cc1_hbm.at[idxb], x1b, sem).wait()

    out = pl.ds(c * K_PER + s * _L, _L)
    pltpu.sync_copy(kbuf, ba_out.at[out])
    pltpu.sync_copy(x0b, x0_out.at[out])
    pltpu.sync_copy(x1b, x1_out.at[out])

    osl = pl.ds(s * _L, _L)
    pltpu.sync_copy(kbuf, sfk.at[osl])
    pltpu.sync_copy(x0b, sf0.at[osl])
    pltpu.sync_copy(x1b, sf1.at[osl])
    plsc.subcore_barrier()

    pltpu.sync_copy(sfk, fk_loc)
    pltpu.sync_copy(sf0, f0_loc)
    pltpu.sync_copy(sf1, f1_loc)

    zero = jnp.zeros((_L,), jnp.float32)
    for i in range(HPW // _L):
        tv = tv_ref[pl.ds(i * _L, _L)]
        valid = tv >= 0
        g = jnp.where(valid, tv, 0)
        bag = plsc.load_gather(fk_loc, [g])
        xg0 = plsc.load_gather(f0_loc, [g])
        xg1 = plsc.load_gather(f1_loc, [g])
        hsl = pl.ds(i * _L, _L)
        gb_ref[hsl] = jnp.where(valid, bag, zero)
        g0_ref[hsl] = jnp.where(valid, xg0, zero)
        g1_ref[hsl] = jnp.where(valid, xg1, zero)

    hout = pl.ds(base, HPW)
    pltpu.sync_copy(gb_ref, bag_out.at[hout])
    pltpu.sync_copy(g0_ref, xg0_out.at[hout])
    pltpu.sync_copy(g1_ref, xg1_out.at[hout])


def _sc_argmax(beta, tid, cc0, cc1, n):
    mesh = plsc.VectorSubcoreMesh(core_axis_name="c", subcore_axis_name="s",
                                  num_cores=_NC)
    hpw = n // 2 // _NS
    k_tot = _NC * K_PER
    f = pl.kernel(
        functools.partial(_sc_argmax_body, N=n, HPW=hpw),
        out_type=[jax.ShapeDtypeStruct((k_tot,), jnp.float32)] * 3
        + [jax.ShapeDtypeStruct((n,), jnp.float32)] * 3,
        mesh=mesh,
        compiler_params=pltpu.CompilerParams(needs_layout_passes=False),
        scratch_types=[
            pltpu.VMEM((K_PER * _L,), jnp.float32),
            pltpu.VMEM((K_PER * _L,), jnp.int32),
            pltpu.VMEM((hpw,), jnp.float32),
            pltpu.VMEM((hpw,), jnp.int32),
            pltpu.VMEM((K_PER,), jnp.float32),
            pltpu.VMEM((K_PER,), jnp.int32),
            pltpu.VMEM_SHARED((_NS, K_PER), jnp.float32),
            pltpu.VMEM_SHARED((_NS, K_PER), jnp.int32),
            pltpu.VMEM_SHARED((K_PER,), jnp.float32),
            pltpu.VMEM_SHARED((K_PER,), jnp.float32),
            pltpu.VMEM_SHARED((K_PER,), jnp.float32),
            pltpu.VMEM((K_PER,), jnp.float32),
            pltpu.VMEM((K_PER,), jnp.float32),
            pltpu.VMEM((K_PER,), jnp.float32),
            pltpu.VMEM((hpw,), jnp.float32),
            pltpu.VMEM((hpw,), jnp.float32),
            pltpu.VMEM((hpw,), jnp.float32),
            pltpu.VMEM((_L,), jnp.int32),
            pltpu.VMEM((_L,), jnp.float32),
            pltpu.VMEM((_L,), jnp.float32),
            pltpu.VMEM((_L,), jnp.float32),
            pltpu.SemaphoreType.DMA,
        ],
    )
    return f(beta, tid, cc0, cc1)


def _tc_kernel(rs_ref, cc_ref, tidx_ref,
               bl_ref, tl_ref, c0l_ref, c1l_ref,
               pel_ref, tel_ref, ptl_ref, ttl_ref,
               pp0l_ref, pp1l_ref, tp0l_ref, tp1l_ref, pidt_ref,
               bag_ref, xg0_ref, xg1_ref,
               ba_row_ref, xa0_row_ref, xa1_row_ref,
               out_ref,
               qaeff_ref, acc_ref, accp_ref, ssum_ref,
               *, B, K, NB, N):
    b = pl.program_id(0)
    off = jnp.where(b * B >= rs_ref[1], K_PER, 0)
    ksl = pl.ds(off, K_PER)

    @pl.when(b == 0)
    def _prep():
        ba_r = ba_row_ref[...]
        ex_r = (ba_r > 0.0).astype(jnp.float32)
        atr = _atanh(ba_r * ex_r)
        qaeff_ref[...] = (atr * atr + Q_MIN) * ex_r
        acc_ref[...] = jnp.zeros((1, K), jnp.float32)
        accp_ref[...] = jnp.zeros((8, K), jnp.float32)
        ssum_ref[0] = 0.0
        ssum_ref[1] = 0.0
        ssum_ref[2] = 0.0

    beta = jnp.clip(bl_ref[...], 1e-6, 1.0 - 1e-4)
    tidl = tl_ref[...]
    valid = (tidl >= 0).astype(jnp.float32)
    at = _atanh(beta)
    q = at * at + Q_MIN

    bag = bag_ref[...]
    atg = _atanh(bag)
    qag = (atg * atg + Q_MIN) * valid
    dg0 = c0l_ref[...] - xg0_ref[...]
    dg1 = c1l_ref[...] - xg1_ref[...]
    d2g = dg0 * dg0 + dg1 * dg1
    rwg = jnp.maximum(1.0 - jnp.sqrt(d2g + 1e-9), 0.0)
    attdiag = q * qag * (d2g - rwg)

    te = tel_ref[...]
    ew = jnp.where(te > 10.0, 1.0, (te / 10.0 + 0.1) / 1.1)
    ste = jnp.sqrt(te + 0.001)
    l = jnp.abs(te - pel_ref[...]) / (ste + E_DEN_OFF)
    delta = ste * HUBER_SCALE
    el = jnp.where(l <= delta, 0.5 * l * l, delta * (l - 0.5 * delta))
    dp0 = tp0l_ref[...] - pp0l_ref[...]
    dp1 = tp1l_ref[...] - pp1l_ref[...]
    posl = (dp0 * dp0 + dp1 * dp1) / 100.0
    dtim = ttl_ref[...] * 1e9 - ptl_ref[...]
    timl = dtim * dtim
    cls = (1e-8 / 6.0) * jnp.sum(pidt_ref[...] * pidt_ref[...], axis=0,
                                 keepdims=True)
    wsum = (el + posl + timl + cls) * ew

    pwv = jnp.where(beta > PAYLOAD_REL_THR * bag, beta, 0.0) * valid
    pw2 = jnp.concatenate([pwv, pwv * wsum], axis=0)

    nmask = (tidl < 0).astype(jnp.float32)
    ssum_ref[0] += jnp.sum(nmask * beta)
    ssum_ref[1] += jnp.sum(nmask)
    ssum_ref[2] += jnp.sum(attdiag)

    x0 = cc_ref[:, 0:1]
    x1 = cc_ref[:, 1:2]
    dx0 = x0 - xa0_row_ref[0:1, ksl]
    dx1 = x1 - xa1_row_ref[0:1, ksl]
    d2 = dx0 * dx0 + dx1 * dx1
    rw = jnp.maximum(1.0 - jnp.sqrt(d2 + 1e-9), 0.0)
    p3 = jax.lax.dot_general(q, rw, (((1,), (0,)), ((), ())),
                             preferred_element_type=jnp.float32)
    acc_ref[0:1, ksl] += p3

    tid = tidx_ref[:, 0:1]
    kio = jax.lax.broadcasted_iota(jnp.int32, (1, K_PER), 1)
    mf = (tid == kio).astype(jnp.float32)
    p2 = jax.lax.dot_general(pw2, mf, (((1,), (0,)), ((), ())),
                             preferred_element_type=jnp.float32)
    accp_ref[0:2, ksl] += p2

    @pl.when(b == NB - 1)
    def _fin():
        ba = ba_row_ref[...]
        exists = (ba > 0.0).astype(jnp.float32)
        n_obj = jnp.maximum(jnp.sum(exists), 1.0)
        l_minb = jnp.sum(exists * (1.0 - ba)) / n_obj
        l_pay = jnp.sum(exists * accp_ref[1:2, :]
                        / (accp_ref[0:1, :] + 1e-9)) / n_obj
        pair = (jnp.sum(qaeff_ref[...] * acc_ref[...]) + ssum_ref[2]) / N
        l_noise = S_B * ssum_ref[0] / jnp.maximum(ssum_ref[1], 1.0)
        out_ref[...] = (pair + l_minb + l_noise + l_pay).reshape(1, 1)


def kernel(pred_beta, pred_ccoords, pred_energy, pred_pos, pred_time,
           pred_id, t_idx, t_energy, t_pos, t_time, t_pid, rowsplits):
    n = pred_beta.shape[0]
    n_events = rowsplits.shape[0] - 1
    k_tot = n_events * K_PER
    B = 4096
    NB = n // B

    cc0 = pred_ccoords[:, 0]
    cc1 = pred_ccoords[:, 1]
    ba, xa0, xa1, bag, xg0, xg1 = _sc_argmax(
        pred_beta[:, 0], t_idx[:, 0], cc0, cc1, n)

    row1 = lambda a: a.reshape(1, n)
    hspec = lambda c: pl.BlockSpec((B, c), lambda b: (b, 0))
    lspec = pl.BlockSpec((1, B), lambda b: (0, b))
    rowspec = pl.BlockSpec((1, k_tot), lambda b: (0, 0))
    out = pl.pallas_call(
        functools.partial(_tc_kernel, B=B, K=k_tot, NB=NB, N=n),
        grid=(NB,),
        in_specs=[
            pl.BlockSpec(memory_space=pltpu.SMEM),
            hspec(2), hspec(1),
        ] + [lspec] * 12 + [
            pl.BlockSpec((6, B), lambda b: (0, b)),
        ] + [lspec] * 3 + [rowspec] * 3,
        out_specs=pl.BlockSpec((1, 1), lambda b: (0, 0)),
        out_shape=jax.ShapeDtypeStruct((1, 1), jnp.float32),
        scratch_shapes=[
            pltpu.VMEM((1, k_tot), jnp.float32),
            pltpu.VMEM((1, k_tot), jnp.float32),
            pltpu.VMEM((8, k_tot), jnp.float32),
            pltpu.SMEM((4,), jnp.float32),
        ],
        compiler_params=pltpu.CompilerParams(
            dimension_semantics=("arbitrary",)),
    )(rowsplits, pred_ccoords, t_idx,
      row1(pred_beta[:, 0]), t_idx[:, 0].reshape(1, n),
      row1(cc0), row1(cc1),
      row1(pred_energy[:, 0]), row1(t_energy[:, 0]),
      row1(pred_time[:, 0]), row1(t_time[:, 0]),
      row1(pred_pos[:, 0]), row1(pred_pos[:, 1]),
      row1(t_pos[:, 0]), row1(t_pos[:, 1]),
      pred_id.T,
      row1(bag), row1(xg0), row1(xg1),
      ba.reshape(1, k_tot), xa0.reshape(1, k_tot), xa1.reshape(1, k_tot))
    return out[0, 0]


# speedup vs baseline: 3.2586x; 1.1098x over previous
import functools

import jax
import jax.numpy as jnp
from jax import lax
from jax.experimental import pallas as pl
from jax.experimental.pallas import tpu as pltpu
from jax.experimental.pallas import tpu_sc as plsc

K_PER = 256
Q_MIN = 0.5
S_B = 1.0
HUBER_SCALE = 2.0
E_DEN_OFF = 1.0
PAYLOAD_REL_THR = 0.1

_NC = 2
_NS = 16
_L = 16


def _atanh(x):
    return 0.5 * jnp.log((1.0 + x) / (1.0 - x))


def _sc_argmax_body(beta_hbm, tid_hbm, cc0_hbm, cc1_hbm,
                    ba_out, x0_out, x1_out, bag_out, xg0_out, xg1_out,
                    tkey, targ, bv_ref, tv_ref, mk_ref, mi_ref,
                    shk, shi,
                    fk_loc, fi_loc, f0_loc, f1_loc,
                    gb_ref, g0_ref, g1_ref, sem,
                    *, N, HPW):
    c = lax.axis_index("c")
    s = lax.axis_index("s")
    base = c * (N // 2) + s * HPW
    lane = jnp.arange(_L, dtype=jnp.int32)

    for i in range(K_PER):
        tkey[pl.ds(i * _L, _L)] = jnp.full((_L,), -1.0, jnp.float32)

    pltpu.sync_copy(beta_hbm.at[pl.ds(base, HPW)], bv_ref)
    pltpu.sync_copy(tid_hbm.at[pl.ds(base, HPW)], tv_ref)

    lbase = lane * K_PER
    for i in range(HPW // _L):
        bv = bv_ref[pl.ds(i * _L, _L)]
        bv = jnp.minimum(jnp.maximum(bv, 1e-6), 1.0 - 1e-4)
        tv = tv_ref[pl.ds(i * _L, _L)]
        valid = tv >= 0
        slot = lbase + jnp.where(valid, tv, 0)
        cur = plsc.load_gather(tkey, [slot])
        m = valid & (bv > cur)
        plsc.store_scatter(tkey, [slot], bv, mask=m)
        hidx = jnp.full((_L,), base + i * _L, jnp.int32) + lane
        plsc.store_scatter(targ, [slot], hidx, mask=m)

    for j in range(K_PER // _L):
        ka = tkey[pl.ds(j * _L, _L)]
        ia = targ[pl.ds(j * _L, _L)]
        for l in range(1, _L):
            o = l * K_PER + j * _L
            kv = tkey[pl.ds(o, _L)]
            iv = targ[pl.ds(o, _L)]
            better = (kv > ka) | ((kv == ka) & (iv < ia))
            ka = jnp.where(better, kv, ka)
            ia = jnp.where(better, iv, ia)
        mk_ref[pl.ds(j * _L, _L)] = ka
        mi_ref[pl.ds(j * _L, _L)] = jnp.where(ka > 0.0, ia, 0)

    pltpu.sync_copy(mk_ref, shk.at[pl.ds(s * K_PER, K_PER)])
    pltpu.sync_copy(mi_ref, shi.at[pl.ds(s * K_PER, K_PER)])
    plsc.subcore_barrier()

    pltpu.sync_copy(shk, tkey)
    pltpu.sync_copy(shi, targ)
    for j in range(K_PER // _L):
        ka = tkey[pl.ds(j * _L, _L)]
        ia = targ[pl.ds(j * _L, _L)]
        for w in range(1, _NS):
            o = w * K_PER + j * _L
            kv = tkey[pl.ds(o, _L)]
            iv = targ[pl.ds(o, _L)]
            better = (kv > ka) | ((kv == ka) & (iv < ia))
            ka = jnp.where(better, kv, ka)
            ia = jnp.where(better, iv, ia)
        fk_loc[pl.ds(j * _L, _L)] = ka
        fi_loc[pl.ds(j * _L, _L)] = jnp.where(ka > 0.0, ia, 0)

    h = K_PER // 2
    ga = pltpu.async_copy(cc0_hbm.at[fi_loc.at[pl.ds(0, h)]],
                          f0_loc.at[pl.ds(0, h)], sem)
    gb = pltpu.async_copy(cc0_hbm.at[fi_loc.at[pl.ds(h, h)]],
                          f0_loc.at[pl.ds(h, h)], sem)
    gc = pltpu.async_copy(cc1_hbm.at[fi_loc.at[pl.ds(0, h)]],
                          f1_loc.at[pl.ds(0, h)], sem)
    gd = pltpu.async_copy(cc1_hbm.at[fi_loc.at[pl.ds(h, h)]],
                          f1_loc.at[pl.ds(h, h)], sem)
    ga.wait()
    gb.wait()
    gc.wait()
    gd.wait()

    osl = pl.ds(s * _L, _L)
    out = pl.ds(c * K_PER + s * _L, _L)
    pltpu.sync_copy(fk_loc.at[osl], ba_out.at[out])
    pltpu.sync_copy(f0_loc.at[osl], x0_out.at[out])
    pltpu.sync_copy(f1_loc.at[osl], x1_out.at[out])

    zero = jnp.zeros((_L,), jnp.float32)
    for i in range(HPW // _L):
        tv = tv_ref[pl.ds(i * _L, _L)]
        valid = tv >= 0
        g = jnp.where(valid, tv, 0)
        bag = plsc.load_gather(fk_loc, [g])
        xg0 = plsc.load_gather(f0_loc, [g])
        xg1 = plsc.load_gather(f1_loc, [g])
        hsl = pl.ds(i * _L, _L)
        gb_ref[hsl] = jnp.where(valid, bag, zero)
        g0_ref[hsl] = jnp.where(valid, xg0, zero)
        g1_ref[hsl] = jnp.where(valid, xg1, zero)

    hout = pl.ds(base, HPW)
    pltpu.sync_copy(gb_ref, bag_out.at[hout])
    pltpu.sync_copy(g0_ref, xg0_out.at[hout])
    pltpu.sync_copy(g1_ref, xg1_out.at[hout])


def _sc_argmax(beta, tid, cc0, cc1, n):
    mesh = plsc.VectorSubcoreMesh(core_axis_name="c", subcore_axis_name="s",
                                  num_cores=_NC)
    hpw = n // 2 // _NS
    k_tot = _NC * K_PER
    f = pl.kernel(
        functools.partial(_sc_argmax_body, N=n, HPW=hpw),
        out_type=[jax.ShapeDtypeStruct((k_tot,), jnp.float32)] * 3
        + [jax.ShapeDtypeStruct((n,), jnp.float32)] * 3,
        mesh=mesh,
        compiler_params=pltpu.CompilerParams(needs_layout_passes=False),
        scratch_types=[
            pltpu.VMEM((K_PER * _L,), jnp.float32),
            pltpu.VMEM((K_PER * _L,), jnp.int32),
            pltpu.VMEM((hpw,), jnp.float32),
            pltpu.VMEM((hpw,), jnp.int32),
            pltpu.VMEM((K_PER,), jnp.float32),
            pltpu.VMEM((K_PER,), jnp.int32),
            pltpu.VMEM_SHARED((_NS * K_PER,), jnp.float32),
            pltpu.VMEM_SHARED((_NS * K_PER,), jnp.int32),
            pltpu.VMEM((K_PER,), jnp.float32),
            pltpu.VMEM((K_PER,), jnp.int32),
            pltpu.VMEM((K_PER,), jnp.float32),
            pltpu.VMEM((K_PER,), jnp.float32),
            pltpu.VMEM((hpw,), jnp.float32),
            pltpu.VMEM((hpw,), jnp.float32),
            pltpu.VMEM((hpw,), jnp.float32),
            pltpu.SemaphoreType.DMA,
        ],
    )
    return f(beta, tid, cc0, cc1)


def _tc_kernel(rs_ref, cc_ref, tidx_ref,
               bl_ref, tl_ref, c0l_ref, c1l_ref,
               pel_ref, tel_ref, ptl_ref, ttl_ref,
               pp0l_ref, pp1l_ref, tp0l_ref, tp1l_ref, pidt_ref,
               bag_ref, xg0_ref, xg1_ref,
               ba_row_ref, xa0_row_ref, xa1_row_ref,
               out_ref,
               qaeff_ref, acc_ref, accp_ref, ssum_ref,
               *, B, K, NB, N):
    b = pl.program_id(0)
    off = jnp.where(b * B >= rs_ref[1], K_PER, 0)
    ksl = pl.ds(off, K_PER)

    @pl.when(b == 0)
    def _prep():
        ba_r = ba_row_ref[...]
        ex_r = (ba_r > 0.0).astype(jnp.float32)
        atr = _atanh(ba_r * ex_r)
        qaeff_ref[...] = (atr * atr + Q_MIN) * ex_r
        acc_ref[...] = jnp.zeros((1, K), jnp.float32)
        accp_ref[...] = jnp.zeros((8, K), jnp.float32)
        ssum_ref[0] = 0.0
        ssum_ref[1] = 0.0
        ssum_ref[2] = 0.0

    beta = jnp.clip(bl_ref[...], 1e-6, 1.0 - 1e-4)
    tidl = tl_ref[...]
    valid = (tidl >= 0).astype(jnp.float32)
    at = _atanh(beta)
    q = at * at + Q_MIN

    bag = bag_ref[...]
    atg = _atanh(bag)
    qag = (atg * atg + Q_MIN) * valid
    dg0 = c0l_ref[...] - xg0_ref[...]
    dg1 = c1l_ref[...] - xg1_ref[...]
    d2g = dg0 * dg0 + dg1 * dg1
    rwg = jnp.maximum(1.0 - jnp.sqrt(d2g + 1e-9), 0.0)
    attdiag = q * qag * (d2g - rwg)

    te = tel_ref[...]
    ew = jnp.where(te > 10.0, 1.0, (te / 10.0 + 0.1) / 1.1)
    ste = jnp.sqrt(te + 0.001)
    l = jnp.abs(te - pel_ref[...]) / (ste + E_DEN_OFF)
    delta = ste * HUBER_SCALE
    el = jnp.where(l <= delta, 0.5 * l * l, delta * (l - 0.5 * delta))
    dp0 = tp0l_ref[...] - pp0l_ref[...]
    dp1 = tp1l_ref[...] - pp1l_ref[...]
    posl = (dp0 * dp0 + dp1 * dp1) / 100.0
    dtim = ttl_ref[...] * 1e9 - ptl_ref[...]
    timl = dtim * dtim
    cls = (1e-8 / 6.0) * jnp.sum(pidt_ref[...] * pidt_ref[...], axis=0,
                                 keepdims=True)
    wsum = (el + posl + timl + cls) * ew

    pwv = jnp.where(beta > PAYLOAD_REL_THR * bag, beta, 0.0) * valid
    pw2 = jnp.concatenate([pwv, pwv * wsum], axis=0)

    nmask = (tidl < 0).astype(jnp.float32)
    ssum_ref[0] += jnp.sum(nmask * beta)
    ssum_ref[1] += jnp.sum(nmask)
    ssum_ref[2] += jnp.sum(attdiag)

    x0 = cc_ref[:, 0:1]
    x1 = cc_ref[:, 1:2]
    dx0 = x0 - xa0_row_ref[0:1, ksl]
    dx1 = x1 - xa1_row_ref[0:1, ksl]
    d2 = dx0 * dx0 + dx1 * dx1
    rw = jnp.maximum(1.0 - jnp.sqrt(d2 + 1e-9), 0.0)
    p3 = jax.lax.dot_general(q, rw, (((1,), (0,)), ((), ())),
                             preferred_element_type=jnp.float32)
    acc_ref[0:1, ksl] += p3

    tid = tidx_ref[:, 0:1]
    kio = jax.lax.broadcasted_iota(jnp.int32, (1, K_PER), 1)
    mf = (tid == kio).astype(jnp.float32)
    p2 = jax.lax.dot_general(pw2, mf, (((1,), (0,)), ((), ())),
                             preferred_element_type=jnp.float32)
    accp_ref[0:2, ksl] += p2

    @pl.when(b == NB - 1)
    def _fin():
        ba = ba_row_ref[...]
        exists = (ba > 0.0).astype(jnp.float32)
        n_obj = jnp.maximum(jnp.sum(exists), 1.0)
        l_minb = jnp.sum(exists * (1.0 - ba)) / n_obj
        l_pay = jnp.sum(exists * accp_ref[1:2, :]
                        / (accp_ref[0:1, :] + 1e-9)) / n_obj
        pair = (jnp.sum(qaeff_ref[...] * acc_ref[...]) + ssum_ref[2]) / N
        l_noise = S_B * ssum_ref[0] / jnp.maximum(ssum_ref[1], 1.0)
        out_ref[...] = (pair + l_minb + l_noise + l_pay).reshape(1, 1)


def kernel(pred_beta, pred_ccoords, pred_energy, pred_pos, pred_time,
           pred_id, t_idx, t_energy, t_pos, t_time, t_pid, rowsplits):
    n = pred_beta.shape[0]
    n_events = rowsplits.shape[0] - 1
    k_tot = n_events * K_PER
    B = 4096
    NB = n // B

    cc0 = pred_ccoords[:, 0]
    cc1 = pred_ccoords[:, 1]
    ba, xa0, xa1, bag, xg0, xg1 = _sc_argmax(
        pred_beta[:, 0], t_idx[:, 0], cc0, cc1, n)

    row1 = lambda a: a.reshape(1, n)
    hspec = lambda c: pl.BlockSpec((B, c), lambda b: (b, 0))
    lspec = pl.BlockSpec((1, B), lambda b: (0, b))
    rowspec = pl.BlockSpec((1, k_tot), lambda b: (0, 0))
    out = pl.pallas_call(
        functools.partial(_tc_kernel, B=B, K=k_tot, NB=NB, N=n),
        grid=(NB,),
        in_specs=[
            pl.BlockSpec(memory_space=pltpu.SMEM),
            hspec(2), hspec(1),
        ] + [lspec] * 12 + [
            pl.BlockSpec((6, B), lambda b: (0, b)),
        ] + [lspec] * 3 + [rowspec] * 3,
        out_specs=pl.BlockSpec((1, 1), lambda b: (0, 0)),
        out_shape=jax.ShapeDtypeStruct((1, 1), jnp.float32),
        scratch_shapes=[
            pltpu.VMEM((1, k_tot), jnp.float32),
            pltpu.VMEM((1, k_tot), jnp.float32),
            pltpu.VMEM((8, k_tot), jnp.float32),
            pltpu.SMEM((4,), jnp.float32),
        ],
        compiler_params=pltpu.CompilerParams(
            dimension_semantics=("arbitrary",)),
    )(rowsplits, pred_ccoords, t_idx,
      row1(pred_beta[:, 0]), t_idx[:, 0].reshape(1, n),
      row1(cc0), row1(cc1),
      row1(pred_energy[:, 0]), row1(t_energy[:, 0]),
      row1(pred_time[:, 0]), row1(t_time[:, 0]),
      row1(pred_pos[:, 0]), row1(pred_pos[:, 1]),
      row1(t_pos[:, 0]), row1(t_pos[:, 1]),
      pred_id.T,
      row1(bag), row1(xg0), row1(xg1),
      ba.reshape(1, k_tot), xa0.reshape(1, k_tot), xa1.reshape(1, k_tot))
    return out[0, 0]
